# Initial kernel scaffold; baseline (speedup 1.0000x reference)
import jax
import jax.numpy as jnp
from jax.experimental import pallas as pl


def kernel(h, bond_idx, angle_idx, torsion_idx, mol_id, onefour_idx, nonbonded_idx, W_bond0, b_bond0, W_angle0, b_angle0, W_torsion0, b_torsion0, W_atom1, b_atom1, W_atom2, b_atom2, W_bond1, b_bond1, W_bond2, b_bond2, W_angle1, b_angle1, W_angle2, b_angle2, W_torsion1, b_torsion1, W_torsion2, b_torsion2, W_mol1, b_mol1, W_mol2, b_mol2):
    raise NotImplementedError("write your pallas kernel here")


# speedup vs baseline: 7.4638x; 7.4638x over previous
import jax
import jax.numpy as jnp
from jax import lax
from jax.experimental import pallas as pl
from jax.experimental.pallas import tpu as pltpu
from jax.experimental.pallas import tpu_sc as plsc

D = 128
N_MOL_CONST = 2500
NW = 32
NC = 2
GCHUNK = 128
GFIRE = 8
SUPER = GCHUNK * GFIRE
SEG = NW * SUPER
ATOM_CHUNK = 128


def _ceil_to(x, m):
    return (x + m - 1) // m * m


def _pad_rows(x, n):
    return jnp.pad(x, ((0, n - x.shape[0]),) + ((0, 0),) * (x.ndim - 1))


def _build_tables(h_pad, qe, wm, c_atom2, n_pad):
    bm = 512
    grid = n_pad // bm

    def body(h_ref, qe_ref, wm_ref, ca_ref, tb, taa, tab, tta, ttb, ske, ke, pm):
        hb = h_ref[...]
        pe = jnp.dot(hb, qe_ref[...], preferred_element_type=jnp.float32)
        pm[...] = jnp.dot(hb, wm_ref[...], preferred_element_type=jnp.float32)
        kev = jnp.abs(pe[:, 0:2] + ca_ref[...])
        ke[...] = kev
        ske[...] = jnp.concatenate([jnp.sqrt(kev[:, 0:1]), kev[:, 1:2]], axis=1)
        tb[...] = pe[:, 2:4]
        taa[...] = pe[:, 4:6]
        tab[...] = pe[:, 6:8]
        tta[...] = pe[:, 8:10]
        ttb[...] = pe[:, 10:12]

    two = jax.ShapeDtypeStruct((n_pad, 2), jnp.float32)
    outs = pl.pallas_call(
        body,
        grid=(grid,),
        in_specs=[
            pl.BlockSpec((bm, D), lambda i: (i, 0)),
            pl.BlockSpec((D, 16), lambda i: (0, 0)),
            pl.BlockSpec((D, D), lambda i: (0, 0)),
            pl.BlockSpec((1, 2), lambda i: (0, 0)),
        ],
        out_specs=[pl.BlockSpec((bm, 2), lambda i: (i, 0))] * 7
        + [pl.BlockSpec((bm, D), lambda i: (i, 0))],
        out_shape=[two] * 7 + [jax.ShapeDtypeStruct((n_pad, D), jnp.float32)],
    )(h_pad, qe, wm, c_atom2)
    return outs


def _sc_gather_scatter(tables, idxs, pm, molid3, zacc, n_pad, n_mol):
    chunks_per_tile = n_pad // (NW * ATOM_CHUNK)
    rows_per_tile = chunks_per_tile * ATOM_CHUNK
    per_tiles = [i.shape[0] // NW for i in idxs]
    max_pt = max(per_tiles)
    mesh = plsc.VectorSubcoreMesh(core_axis_name="c", subcore_axis_name="s")

    def body(tb, taa, tab, tta, ttb, tske,
             ib, iaa, iab, ita, itb, iske,
             pm_ref, mid3_ref, z_ref,
             m2_out, gb, gaa, gab, gta, gtb, gske,
             idx_v, rows_v, mid_v, mrow_v, acc, sem):
        cid = lax.axis_index("c")
        sid = lax.axis_index("s")
        wid = sid * NC + cid

        @pl.when(sid == 0)
        def _():
            pltpu.sync_copy(z_ref, acc)

        plsc.subcore_barrier()
        pltpu.sync_copy(mid3_ref.at[wid], mid_v)

        @pl.loop(0, chunks_per_tile)
        def _(j):
            base = wid * rows_per_tile + j * ATOM_CHUNK
            pltpu.sync_copy(pm_ref.at[pl.ds(base, ATOM_CHUNK)], mrow_v)
            pltpu.sync_copy(mrow_v, acc.at[mid_v.at[j]], add=True)

        plsc.subcore_barrier()

        @pl.when(sid == 0)
        def _():
            pltpu.sync_copy(acc, m2_out.at[cid])

        for tbl, idx, out, pt in (
            (tb, ib, gb, per_tiles[0]),
            (taa, iaa, gaa, per_tiles[1]),
            (tab, iab, gab, per_tiles[2]),
            (tta, ita, gta, per_tiles[3]),
            (ttb, itb, gtb, per_tiles[4]),
            (tske, iske, gske, per_tiles[5]),
        ):
            base = wid * pt
            pltpu.sync_copy(idx.at[pl.ds(base, pt)], idx_v.at[pl.ds(0, pt)])

            @pl.loop(0, pt // SUPER)
            def _(s):
                descs = []
                for b in range(GFIRE):
                    descs.append(pltpu.async_copy(
                        tbl.at[idx_v.at[pl.ds(s * SUPER + b * GCHUNK, GCHUNK)]],
                        rows_v.at[pl.ds(b * GCHUNK, GCHUNK)], sem))
                for dsc in descs:
                    dsc.wait()
                pltpu.sync_copy(rows_v, out.at[pl.ds(base + s * SUPER, SUPER)])

    out_type = (
        jax.ShapeDtypeStruct((NC, n_mol, D), jnp.float32),
    ) + tuple(jax.ShapeDtypeStruct((i.shape[0], 2), jnp.float32) for i in idxs)
    scratch = [
        pltpu.VMEM((max_pt,), jnp.int32),
        pltpu.VMEM((SUPER, 2), jnp.float32),
        pltpu.VMEM((chunks_per_tile, ATOM_CHUNK), jnp.int32),
        pltpu.VMEM((ATOM_CHUNK, D), jnp.float32),
        pltpu.VMEM_SHARED((n_mol, D), jnp.float32),
        pltpu.SemaphoreType.DMA,
    ]
    fn = pl.kernel(
        body, out_type=out_type, mesh=mesh, scratch_types=scratch,
        compiler_params=pltpu.CompilerParams(use_tc_tiling_on_sc=False))
    return fn(*tables, *idxs, pm, molid3, zacc)


def _combine_abs(n_out, parts, c2):
    bm = 512
    grid = _ceil_to(n_out, bm) // bm

    def body(*refs):
        ins, cref, out = refs[:-2], refs[-2], refs[-1]
        s = ins[0][...]
        for r in ins[1:]:
            s = s + r[...]
        out[...] = jnp.abs(s + cref[...])

    in_specs = [
        pl.BlockSpec((bm, 2), lambda i, o=off // bm: (i + o, 0))
        for (_, off) in parts
    ] + [pl.BlockSpec((1, 2), lambda i: (0, 0))]
    return pl.pallas_call(
        body,
        grid=(grid,),
        in_specs=in_specs,
        out_specs=pl.BlockSpec((bm, 2), lambda i: (i, 0)),
        out_shape=jax.ShapeDtypeStruct((grid * bm, 2), jnp.float32),
    )(*[a for (a, _) in parts], c2)


def _pair_prod(n_out, src, a_off, b_off):
    bm = 512
    grid = _ceil_to(n_out, bm) // bm

    def body(a_ref, b_ref, out):
        out[...] = a_ref[...] * b_ref[...]

    return pl.pallas_call(
        body,
        grid=(grid,),
        in_specs=[
            pl.BlockSpec((bm, 2), lambda i, o=a_off // bm: (i + o, 0)),
            pl.BlockSpec((bm, 2), lambda i, o=b_off // bm: (i + o, 0)),
        ],
        out_specs=pl.BlockSpec((bm, 2), lambda i: (i, 0)),
        out_shape=jax.ShapeDtypeStruct((grid * bm, 2), jnp.float32),
    )(src, src)


def _mol_mlp(m2, b1, w2, b2, n_mol):
    def body(m_ref, b1_ref, w2_ref, b2_ref, out):
        m = m_ref[0] + m_ref[1] + b1_ref[...]
        u = jnp.dot(jnp.tanh(m), w2_ref[...], preferred_element_type=jnp.float32)
        out[...] = u + b2_ref[...]

    return pl.pallas_call(
        body,
        out_shape=jax.ShapeDtypeStruct((n_mol, 1), jnp.float32),
    )(m2, b1.reshape(1, D), w2, b2.reshape(1, 1))


def kernel(h, bond_idx, angle_idx, torsion_idx, mol_id, onefour_idx,
           nonbonded_idx, W_bond0, b_bond0, W_angle0, b_angle0, W_torsion0,
           b_torsion0, W_atom1, b_atom1, W_atom2, b_atom2, W_bond1, b_bond1,
           W_bond2, b_bond2, W_angle1, b_angle1, W_angle2, b_angle2,
           W_torsion1, b_torsion1, W_torsion2, b_torsion2, W_mol1, b_mol1,
           W_mol2, b_mol2):
    n_atom = h.shape[0]
    n_bond = bond_idx.shape[0]
    n_angle = angle_idx.shape[0]
    n_torsion = torsion_idx.shape[0]
    n_of = onefour_idx.shape[0]
    n_nb = nonbonded_idx.shape[0]
    n_mol = N_MOL_CONST

    r1 = W_bond1 @ W_bond2
    q_bond = (W_bond0[:D] + W_bond0[D:]) @ r1
    c_bond = ((2.0 * b_bond0) @ r1 + b_bond1 @ W_bond2 + b_bond2).reshape(1, 2)
    r1 = W_angle1 @ W_angle2
    qa_ang = (W_angle0[:D] + W_angle0[2 * D:]) @ r1
    qb_ang = (2.0 * W_angle0[D:2 * D]) @ r1
    c_ang = ((2.0 * b_angle0) @ r1 + b_angle1 @ W_angle2 + b_angle2).reshape(1, 2)
    r1 = W_torsion1 @ W_torsion2
    qa_tor = (W_torsion0[:D] + W_torsion0[3 * D:]) @ r1
    qb_tor = (W_torsion0[D:2 * D] + W_torsion0[2 * D:3 * D]) @ r1
    c_tor = ((2.0 * b_torsion0) @ r1 + b_torsion1 @ W_torsion2 + b_torsion2).reshape(1, 2)
    q_atom = W_atom1 @ W_atom2
    c_atom2 = (b_atom1 @ W_atom2 + b_atom2).reshape(1, 2)
    qe = jnp.concatenate(
        [q_atom, q_bond, qa_ang, qb_ang, qa_tor, qb_tor,
         jnp.zeros((D, 4), jnp.float32)], axis=1)

    n_pad = _ceil_to(n_atom, NW * ATOM_CHUNK)
    h_pad = _pad_rows(h, n_pad)
    tb, taa, tab, tta, ttb, ske, ke, pm = _build_tables(
        h_pad, qe, W_mol1, c_atom2, n_pad)

    def seg(x):
        x = x.astype(jnp.int32)
        return jnp.pad(x, (0, _ceil_to(x.shape[0], SEG) - x.shape[0]))

    ib = jnp.concatenate([seg(bond_idx[:, 0]), seg(bond_idx[:, 1])])
    iaa = jnp.concatenate([seg(angle_idx[:, 0]), seg(angle_idx[:, 2])])
    iab = seg(angle_idx[:, 1])
    ita = jnp.concatenate([seg(torsion_idx[:, 0]), seg(torsion_idx[:, 3])])
    itb = jnp.concatenate([seg(torsion_idx[:, 1]), seg(torsion_idx[:, 2])])
    iske = jnp.concatenate([seg(onefour_idx[:, 0]), seg(onefour_idx[:, 1]),
                            seg(nonbonded_idx[:, 0]), seg(nonbonded_idx[:, 1])])
    molid3 = jnp.pad(mol_id.astype(jnp.int32),
                     (0, n_pad - n_atom)).reshape(NW, -1, ATOM_CHUNK)
    zacc = jnp.zeros((n_mol, D), jnp.float32)

    m2, gb, gaa, gab, gta, gtb, gske = _sc_gather_scatter(
        (tb, taa, tab, tta, ttb, ske), (ib, iaa, iab, ita, itb, iske),
        pm, molid3, zacc, n_pad, n_mol)

    hb = ib.shape[0] // 2
    keb = _combine_abs(n_bond, [(gb, 0), (gb, hb)], c_bond)
    ha = iaa.shape[0] // 2
    kea = _combine_abs(n_angle, [(gaa, 0), (gaa, ha), (gab, 0)], c_ang)
    ht = ita.shape[0] // 2
    ket = _combine_abs(
        n_torsion, [(gta, 0), (gta, ht), (gtb, 0), (gtb, ht)], c_tor)
    s_of = _ceil_to(n_of, SEG)
    s_nb = _ceil_to(n_nb, SEG)
    p_of = _pair_prod(n_of, gske, 0, s_of)
    p_nb = _pair_prod(n_nb, gske, 2 * s_of, 2 * s_of + s_nb)
    u0 = _mol_mlp(m2, b_mol1, W_mol2, b_mol2, n_mol)

    return jnp.concatenate([
        ke[:n_atom, 0], ke[:n_atom, 1],
        keb[:n_bond, 0], keb[:n_bond, 1],
        kea[:n_angle, 0], kea[:n_angle, 1],
        ket[:n_torsion, 0], ket[:n_torsion, 1],
        u0[:, 0],
        p_of[:n_of, 0], p_of[:n_of, 1],
        p_nb[:n_nb, 0], p_nb[:n_nb, 1],
    ])


# speedup vs baseline: 15.1738x; 2.0330x over previous
import jax
import jax.numpy as jnp
from jax import lax
from jax.experimental import pallas as pl
from jax.experimental.pallas import tpu as pltpu
from jax.experimental.pallas import tpu_sc as plsc

D = 128
N_MOL_CONST = 2500
NW = 32
NC = 2
GCHUNK = 128
ATOM_CHUNK = 128
TABW = 16

STAGES = {
    'bond': (2, 512, [(0, 2), (1, 2)], 0, 'sum'),
    'angle': (3, 384, [(0, 4), (2, 4), (1, 6)], 2, 'sum'),
    'torsion': (4, 256, [(0, 8), (3, 8), (1, 10), (2, 10)], 4, 'sum'),
    'of': (2, 512, [(0, 0), (1, 0)], None, 'prod'),
    'nb': (2, 512, [(0, 0), (1, 0)], None, 'prod'),
}


def _ceil_to(x, m):
    return (x + m - 1) // m * m


def _build_tables(h_pad, qe, wm, c_atom2, n_pad):
    bm = 512
    grid = n_pad // bm

    def body(h_ref, qe_ref, wm_ref, ca_ref, tab, ka, ea, pm):
        hb = h_ref[...]
        pe = jnp.dot(hb, qe_ref[...], preferred_element_type=jnp.float32)
        pm[...] = jnp.dot(hb, wm_ref[...], preferred_element_type=jnp.float32)
        kev = jnp.abs(pe[:, 0:2] + ca_ref[...])
        ka[...] = kev[:, 0]
        ea[...] = kev[:, 1]
        tab[...] = jnp.concatenate(
            [jnp.sqrt(kev[:, 0:1]), kev[:, 1:2], pe[:, 2:12],
             jnp.zeros((bm, 4), jnp.float32)], axis=1)

    return pl.pallas_call(
        body,
        grid=(grid,),
        in_specs=[
            pl.BlockSpec((bm, D), lambda i: (i, 0)),
            pl.BlockSpec((D, TABW), lambda i: (0, 0)),
            pl.BlockSpec((D, D), lambda i: (0, 0)),
            pl.BlockSpec((1, 2), lambda i: (0, 0)),
        ],
        out_specs=[
            pl.BlockSpec((bm, TABW), lambda i: (i, 0)),
            pl.BlockSpec((bm,), lambda i: (i,)),
            pl.BlockSpec((bm,), lambda i: (i,)),
            pl.BlockSpec((bm, D), lambda i: (i, 0)),
        ],
        out_shape=[
            jax.ShapeDtypeStruct((n_pad, TABW), jnp.float32),
            jax.ShapeDtypeStruct((n_pad,), jnp.float32),
            jax.ShapeDtypeStruct((n_pad,), jnp.float32),
            jax.ShapeDtypeStruct((n_pad, D), jnp.float32),
        ],
    )(h_pad, qe, wm, c_atom2)


def _sc_kernel(tab, idxs, epads, pm, molid3, zacc, cvecs, n_pad, n_mol):
    chunks_per_tile = n_pad // (NW * ATOM_CHUNK)
    rows_per_tile = chunks_per_tile * ATOM_CHUNK
    names = list(STAGES)
    epts = {k: epads[k] // NW for k in names}
    max_idx = max(epts[k] * STAGES[k][0] for k in names)
    max_rows = max(STAGES[k][1] * STAGES[k][0] for k in names)
    mesh = plsc.VectorSubcoreMesh(core_axis_name="c", subcore_axis_name="s")

    def body(tab_ref, ib, ia, it, iof, inb, pm_ref, mid3_ref, z_ref, cv_ref,
             m2_out, okb, oeb, oka, oea, okt, oet, okf, oef, okn, oen,
             idx_v, rows_v, outk_v, oute_v, mid_v, mrow_v, acc, c_v, sem):
        cid = lax.axis_index("c")
        sid = lax.axis_index("s")
        wid = sid * NC + cid
        iota = lax.iota(jnp.int32, 16)

        pltpu.sync_copy(cv_ref, c_v)

        @pl.when(sid == 0)
        def _():
            pltpu.sync_copy(z_ref, acc)

        plsc.subcore_barrier()
        pltpu.sync_copy(mid3_ref.at[wid], mid_v)

        @pl.loop(0, chunks_per_tile)
        def _(j):
            base = wid * rows_per_tile + j * ATOM_CHUNK
            pltpu.sync_copy(pm_ref.at[pl.ds(base, ATOM_CHUNK)], mrow_v)
            pltpu.sync_copy(mrow_v, acc.at[mid_v.at[j]], add=True)

        plsc.subcore_barrier()

        @pl.when(sid == 0)
        def _():
            pltpu.sync_copy(acc, m2_out.at[cid])

        stage_io = {'bond': (ib, okb, oeb), 'angle': (ia, oka, oea),
                    'torsion': (it, okt, oet), 'of': (iof, okf, oef),
                    'nb': (inb, okn, oen)}
        for name in names:
            stride, es, terms, crow, mode = STAGES[name]
            idx_hbm, out_k, out_e = stage_io[name]
            ept = epts[name]
            n_idx = ept * stride
            pltpu.sync_copy(idx_hbm.at[pl.ds(wid * n_idx, n_idx)],
                            idx_v.at[pl.ds(0, n_idx)])
            nch = es * stride // GCHUNK

            @pl.loop(0, ept // es)
            def _(s, _stride=stride, _es=es, _terms=terms, _crow=crow,
                  _mode=mode, _nch=nch, _ept=ept, _ok=out_k, _oe=out_e):
                descs = []
                for ch in range(_nch):
                    off = s * (_es * _stride) + ch * GCHUNK
                    descs.append(pltpu.async_copy(
                        tab_ref.at[idx_v.at[pl.ds(off, GCHUNK)]],
                        rows_v.at[pl.ds(ch * GCHUNK, GCHUNK)], sem))
                for dsc in descs:
                    dsc.wait()
                for oc in range(2):
                    ob = outk_v if oc == 0 else oute_v
                    for g in range(_es // 16):
                        vals = []
                        for (roff, cbase) in _terms:
                            ridx = g * 16 * _stride + _stride * iota + roff
                            cidx = jnp.full((16,), cbase + oc, jnp.int32)
                            vals.append(plsc.load_gather(rows_v, [ridx, cidx]))
                        if _mode == 'sum':
                            v = vals[0]
                            for x in vals[1:]:
                                v = v + x
                            v = jnp.abs(v + c_v[_crow + oc])
                        else:
                            v = vals[0] * vals[1]
                        ob[pl.ds(g * 16, 16)] = v
                pltpu.sync_copy(outk_v.at[pl.ds(0, _es)],
                                _ok.at[pl.ds(wid * _ept + s * _es, _es)])
                pltpu.sync_copy(oute_v.at[pl.ds(0, _es)],
                                _oe.at[pl.ds(wid * _ept + s * _es, _es)])

    one = lambda n: jax.ShapeDtypeStruct((n,), jnp.float32)
    out_type = (jax.ShapeDtypeStruct((NC, n_mol, D), jnp.float32),)
    for k in names:
        out_type += (one(epads[k]), one(epads[k]))
    scratch = [
        pltpu.VMEM((max_idx,), jnp.int32),
        pltpu.VMEM((max_rows, TABW), jnp.float32),
        pltpu.VMEM((512,), jnp.float32),
        pltpu.VMEM((512,), jnp.float32),
        pltpu.VMEM((chunks_per_tile, ATOM_CHUNK), jnp.int32),
        pltpu.VMEM((ATOM_CHUNK, D), jnp.float32),
        pltpu.VMEM_SHARED((n_mol, D), jnp.float32),
        pltpu.VMEM((8, 16), jnp.float32),
        pltpu.SemaphoreType.DMA,
    ]
    fn = pl.kernel(
        body, out_type=out_type, mesh=mesh, scratch_types=scratch,
        compiler_params=pltpu.CompilerParams(
            use_tc_tiling_on_sc=False, needs_layout_passes=False))
    return fn(tab, idxs['bond'], idxs['angle'], idxs['torsion'], idxs['of'],
              idxs['nb'], pm, molid3, zacc, cvecs)


def _mol_mlp(m2, b1, w2, b2, n_mol):
    def body(m_ref, b1_ref, w2_ref, b2_ref, out):
        m = m_ref[0] + m_ref[1] + b1_ref[...]
        u = jnp.dot(jnp.tanh(m), w2_ref[...], preferred_element_type=jnp.float32)
        out[...] = u + b2_ref[...]

    return pl.pallas_call(
        body,
        out_shape=jax.ShapeDtypeStruct((n_mol, 1), jnp.float32),
    )(m2, b1.reshape(1, D), w2, b2.reshape(1, 1))


def kernel(h, bond_idx, angle_idx, torsion_idx, mol_id, onefour_idx,
           nonbonded_idx, W_bond0, b_bond0, W_angle0, b_angle0, W_torsion0,
           b_torsion0, W_atom1, b_atom1, W_atom2, b_atom2, W_bond1, b_bond1,
           W_bond2, b_bond2, W_angle1, b_angle1, W_angle2, b_angle2,
           W_torsion1, b_torsion1, W_torsion2, b_torsion2, W_mol1, b_mol1,
           W_mol2, b_mol2):
    n_atom = h.shape[0]
    n_mol = N_MOL_CONST

    r1 = W_bond1 @ W_bond2
    q_bond = (W_bond0[:D] + W_bond0[D:]) @ r1
    c_bond = (2.0 * b_bond0) @ r1 + b_bond1 @ W_bond2 + b_bond2
    r1 = W_angle1 @ W_angle2
    qa_ang = (W_angle0[:D] + W_angle0[2 * D:]) @ r1
    qb_ang = (2.0 * W_angle0[D:2 * D]) @ r1
    c_ang = (2.0 * b_angle0) @ r1 + b_angle1 @ W_angle2 + b_angle2
    r1 = W_torsion1 @ W_torsion2
    qa_tor = (W_torsion0[:D] + W_torsion0[3 * D:]) @ r1
    qb_tor = (W_torsion0[D:2 * D] + W_torsion0[2 * D:3 * D]) @ r1
    c_tor = (2.0 * b_torsion0) @ r1 + b_torsion1 @ W_torsion2 + b_torsion2
    q_atom = W_atom1 @ W_atom2
    c_atom2 = (b_atom1 @ W_atom2 + b_atom2).reshape(1, 2)
    qe = jnp.concatenate(
        [q_atom, q_bond, qa_ang, qb_ang, qa_tor, qb_tor,
         jnp.zeros((D, 4), jnp.float32)], axis=1)
    cvecs = jnp.concatenate(
        [jnp.tile(c_bond.reshape(2, 1), (1, 16)),
         jnp.tile(c_ang.reshape(2, 1), (1, 16)),
         jnp.tile(c_tor.reshape(2, 1), (1, 16)),
         jnp.zeros((2, 16), jnp.float32)], axis=0)

    n_pad = _ceil_to(n_atom, NW * ATOM_CHUNK)
    h_pad = jnp.pad(h, ((0, n_pad - n_atom), (0, 0)))
    tab, ka, ea, pm = _build_tables(h_pad, qe, W_mol1, c_atom2, n_pad)

    raw = {'bond': bond_idx, 'angle': angle_idx, 'torsion': torsion_idx,
           'of': onefour_idx, 'nb': nonbonded_idx}
    idxs, epads = {}, {}
    for name, arr in raw.items():
        stride, es, _, _, _ = STAGES[name]
        epad = _ceil_to(arr.shape[0], NW * es)
        flat = arr.astype(jnp.int32).reshape(-1)
        idxs[name] = jnp.pad(flat, (0, epad * stride - flat.shape[0]))
        epads[name] = epad
    molid3 = jnp.pad(mol_id.astype(jnp.int32),
                     (0, n_pad - n_atom)).reshape(NW, -1, ATOM_CHUNK)
    zacc = jnp.zeros((n_mol, D), jnp.float32)

    (m2, kb, eb, kan, ean, kt, et, kf, ef, kn, en) = _sc_kernel(
        tab, idxs, epads, pm, molid3, zacc, cvecs, n_pad, n_mol)

    u0 = _mol_mlp(m2, b_mol1, W_mol2, b_mol2, n_mol)

    n_bond = bond_idx.shape[0]
    n_angle = angle_idx.shape[0]
    n_torsion = torsion_idx.shape[0]
    n_of = onefour_idx.shape[0]
    n_nb = nonbonded_idx.shape[0]
    return jnp.concatenate([
        ka[:n_atom], ea[:n_atom],
        kb[:n_bond], eb[:n_bond],
        kan[:n_angle], ean[:n_angle],
        kt[:n_torsion], et[:n_torsion],
        u0[:, 0],
        kf[:n_of], ef[:n_of],
        kn[:n_nb], en[:n_nb],
    ])
